# baseline (speedup 1.0000x reference)
import functools

import jax
import jax.numpy as jnp
from jax import lax
from jax.experimental import pallas as pl
from jax.experimental.pallas import tpu as pltpu
from jax.experimental.pallas import tpu_sc as plsc

_NC = 2
_NS = 16
_NW = _NC * _NS
_LANES = 16
_CHUNK = 120


def _msg_body(x_ref, w_ref, b_ref, o_ref):
    m = jnp.tanh(
        jnp.dot(x_ref[...], w_ref[...], preferred_element_type=jnp.float32)
        + b_ref[...])
    o_ref[...] = jnp.maximum(m, 0.0)


def _tc_msg(x, w, b, bn):
    n, d = x.shape
    return pl.pallas_call(
        _msg_body,
        grid=(n // bn,),
        in_specs=[
            pl.BlockSpec((bn, d), lambda i: (i, 0)),
            pl.BlockSpec((d, d), lambda i: (0, 0)),
            pl.BlockSpec((1, d), lambda i: (0, 0)),
        ],
        out_specs=pl.BlockSpec((bn, d), lambda i: (i, 0)),
        out_shape=jax.ShapeDtypeStruct((n, d), jnp.float32),
    )(x, w, b.reshape(1, d))


def _upd_body(with_psum, x_ref, a0_ref, a1_ref, w1_ref, b1_ref,
              w2_ref, b2_ref, g_ref, be_ref, *out_refs):
    x = x_ref[...]
    agg = a0_ref[...] + a1_ref[...]
    h = jnp.concatenate([x, agg], axis=1)
    h = jnp.tanh(
        jnp.dot(h, w1_ref[...], preferred_element_type=jnp.float32)
        + b1_ref[...])
    h = jnp.dot(h, w2_ref[...], preferred_element_type=jnp.float32) + b2_ref[...]
    mu = jnp.mean(h, axis=-1, keepdims=True)
    c = h - mu
    var = jnp.mean(c * c, axis=-1, keepdims=True)
    xn = g_ref[...] * c / jnp.sqrt(var + 1e-5) + be_ref[...]
    out_refs[0][...] = xn
    if with_psum:
        out_refs[1][...] = jnp.sum(xn, axis=0)[None, None, :]


def _tc_update(x, a0, a1, w1, b1, w2, b2, ln_g, ln_b, bn, with_psum):
    n, d = x.shape
    grid = n // bn
    out_shape = [jax.ShapeDtypeStruct((n, d), jnp.float32)]
    out_specs = [pl.BlockSpec((bn, d), lambda i: (i, 0))]
    if with_psum:
        out_shape.append(jax.ShapeDtypeStruct((grid, 1, d), jnp.float32))
        out_specs.append(pl.BlockSpec((1, 1, d), lambda i: (i, 0, 0)))
    res = pl.pallas_call(
        functools.partial(_upd_body, with_psum),
        grid=(grid,),
        in_specs=[
            pl.BlockSpec((bn, d), lambda i: (i, 0)),
            pl.BlockSpec((bn, d), lambda i: (i, 0)),
            pl.BlockSpec((bn, d), lambda i: (i, 0)),
            pl.BlockSpec((2 * d, d), lambda i: (0, 0)),
            pl.BlockSpec((1, d), lambda i: (0, 0)),
            pl.BlockSpec((d, d), lambda i: (0, 0)),
            pl.BlockSpec((1, d), lambda i: (0, 0)),
            pl.BlockSpec((1, d), lambda i: (0, 0)),
            pl.BlockSpec((1, d), lambda i: (0, 0)),
        ],
        out_specs=out_specs,
        out_shape=out_shape,
    )(x, a0, a1, w1, b1.reshape(1, d), w2, b2.reshape(1, d),
      ln_g.reshape(1, d), ln_b.reshape(1, d))
    if with_psum:
        return res[0], res[1]
    return res[0], None


def _heads_body(n_total, x_ref, ps_ref, gf_ref, wg_ref, bg_ref,
                wm1_ref, bm1_ref, wm2_ref, bm2_ref,
                wh1_ref, bh1_ref, wh2_ref, bh2_ref,
                wa1_ref, ba1_ref, wa2_ref, ba2_ref,
                om_ref, oh_ref, oa_ref):
    total = jnp.sum(ps_ref[...], axis=(0, 1))[None, :] * (1.0 / n_total)
    g = jnp.tanh(
        jnp.dot(total, wg_ref[...], preferred_element_type=jnp.float32)
        + bg_ref[...] + gf_ref[...])
    x = x_ref[...]
    feat = jnp.concatenate(
        [x, jnp.broadcast_to(g, (x.shape[0], g.shape[1]))], axis=1)
    hm = jnp.tanh(
        jnp.dot(feat, wm1_ref[...], preferred_element_type=jnp.float32)
        + bm1_ref[...])
    om_ref[...] = jnp.tanh(
        jnp.dot(hm, wm2_ref[...], preferred_element_type=jnp.float32)
        + bm2_ref[...])
    hh = jnp.tanh(
        jnp.dot(feat, wh1_ref[...], preferred_element_type=jnp.float32)
        + bh1_ref[...])
    oh_ref[...] = jnp.tanh(
        jnp.dot(hh, wh2_ref[...], preferred_element_type=jnp.float32)
        + bh2_ref[...])
    ha = jnp.tanh(
        jnp.dot(feat, wa1_ref[...], preferred_element_type=jnp.float32)
        + ba1_ref[...])
    oa_ref[...] = jnp.tanh(
        jnp.dot(ha, wa2_ref[...], preferred_element_type=jnp.float32)
        + ba2_ref[...])


def _tc_heads(x, psum, gf, wg, bg, wm1, bm1, wm2, bm2, wh1, bh1, wh2, bh2,
              wa1, ba1, wa2, ba2, bn):
    n, d = x.shape
    grid = n // bn
    nb = psum.shape[0]
    g_dim = gf.shape[1]
    h_dim = wm1.shape[1]
    dg = d + g_dim
    move_w = wm2.shape[1]
    wspec = lambda a, b: pl.BlockSpec((a, b), lambda i: (0, 0))
    return pl.pallas_call(
        functools.partial(_heads_body, n),
        grid=(grid,),
        in_specs=[
            pl.BlockSpec((bn, d), lambda i: (i, 0)),
            pl.BlockSpec((nb, 1, d), lambda i: (0, 0, 0)),
            wspec(1, g_dim), wspec(d, g_dim), wspec(1, g_dim),
            wspec(dg, h_dim), wspec(1, h_dim), wspec(h_dim, move_w),
            wspec(1, move_w),
            wspec(dg, h_dim), wspec(1, h_dim), wspec(h_dim, 1), wspec(1, 1),
            wspec(dg, h_dim), wspec(1, h_dim), wspec(h_dim, 1), wspec(1, 1),
        ],
        out_specs=[
            pl.BlockSpec((bn, move_w), lambda i: (i, 0)),
            pl.BlockSpec((bn, 1), lambda i: (i, 0)),
            pl.BlockSpec((bn, 1), lambda i: (i, 0)),
        ],
        out_shape=[
            jax.ShapeDtypeStruct((n, move_w), jnp.float32),
            jax.ShapeDtypeStruct((n, 1), jnp.float32),
            jax.ShapeDtypeStruct((n, 1), jnp.float32),
        ],
    )(x, psum, gf, wg, bg.reshape(1, g_dim),
      wm1, bm1.reshape(1, h_dim), wm2, bm2.reshape(1, move_w),
      wh1, bh1.reshape(1, h_dim), wh2, bh2.reshape(1, 1),
      wa1, ba1.reshape(1, h_dim), wa2, ba2.reshape(1, 1))


_NBUF = 3
_IRING = 6


def _sc_edge_agg(msg, src1d, dst1d, zeros_tile):
    n, d = msg.shape
    n_chunks = src1d.shape[0] // _CHUNK
    cpw = n_chunks // _NW
    rows_per_tile = ((n // _NS + 7) // 8) * 8
    n_pad = rows_per_tile * _NS
    mesh = plsc.VectorSubcoreMesh(core_axis_name="c", subcore_axis_name="s")

    @functools.partial(
        pl.kernel,
        out_type=jax.ShapeDtypeStruct((_NC * n_pad, d), jnp.float32),
        mesh=mesh,
        scratch_types=[
            pltpu.VMEM((_IRING, _CHUNK), jnp.int32),
            pltpu.VMEM((_IRING, _CHUNK), jnp.int32),
            pltpu.VMEM((_NBUF, _CHUNK, d), jnp.float32),
            pltpu.VMEM_SHARED((n_pad, d), jnp.float32),
            pltpu.SemaphoreType.DMA((_IRING,)),
            pltpu.SemaphoreType.DMA((_NBUF,)),
            pltpu.SemaphoreType.DMA((_NBUF,)),
        ],
        compiler_params=pltpu.CompilerParams(needs_layout_passes=False),
    )
    def k(msg_hbm, src_hbm, dst_hbm, zero_hbm, out_hbm, sidx, didx, rows, acc,
          isem, gsem, ssem):
        c = lax.axis_index("c")
        s = lax.axis_index("s")
        wid = s * _NC + c
        base0 = wid * cpw * _CHUNK

        def prefetch_idx(t):
            @pl.when(t < cpw)
            def _():
                sl = lax.rem(t, _IRING)
                base = base0 + t * _CHUNK
                pltpu.async_copy(src_hbm.at[pl.ds(base, _CHUNK)],
                                 sidx.at[sl], isem.at[sl])
                pltpu.async_copy(dst_hbm.at[pl.ds(base, _CHUNK)],
                                 didx.at[sl], isem.at[sl])

        prefetch_idx(0)
        prefetch_idx(1)
        pltpu.sync_copy(zero_hbm, acc.at[pl.ds(s * rows_per_tile, rows_per_tile)])
        plsc.subcore_barrier()

        def step(j, carry):
            @pl.when(j < cpw)
            def _():
                b = lax.rem(j, _NBUF)
                sl = lax.rem(j, _IRING)

                @pl.when(j >= _NBUF)
                def _():
                    pltpu.make_async_copy(rows.at[b], acc.at[didx.at[0]],
                                          ssem.at[b]).wait()

                pltpu.make_async_copy(src_hbm.at[pl.ds(0, _CHUNK)],
                                      sidx.at[sl], isem.at[sl]).wait()
                pltpu.make_async_copy(dst_hbm.at[pl.ds(0, _CHUNK)],
                                      didx.at[sl], isem.at[sl]).wait()
                pltpu.async_copy(msg_hbm.at[sidx.at[sl]], rows.at[b],
                                 gsem.at[b])
                prefetch_idx(j + 2)

            jj = j - 1

            @pl.when((jj >= 0) & (jj < cpw))
            def _():
                bb = lax.rem(jj, _NBUF)
                sl2 = lax.rem(jj, _IRING)
                pltpu.make_async_copy(msg_hbm.at[sidx.at[0]], rows.at[bb],
                                      gsem.at[bb]).wait()
                pltpu.async_copy(rows.at[bb], acc.at[didx.at[sl2]],
                                 ssem.at[bb], add=True)

            return carry

        lax.fori_loop(0, cpw + 1, step, 0)
        for b in range(_NBUF):
            pltpu.make_async_copy(rows.at[b], acc.at[didx.at[0]],
                                  ssem.at[b]).wait()
        plsc.subcore_barrier()
        pltpu.sync_copy(
            acc.at[pl.ds(s * rows_per_tile, rows_per_tile)],
            out_hbm.at[pl.ds(c * n_pad + s * rows_per_tile, rows_per_tile)])

    return k(msg, src1d, dst1d, zeros_tile), n_pad


def _sc_heads_gather(mtab, htab, atab, ally_pad, adst_pad, move_w):
    tn = htab.shape[0]
    apad = ally_pad.shape[0]
    epad = adst_pad.shape[0]
    ept = epad // _NW
    mpt = apad * move_w // _NW
    hpt = apad // _NW
    mesh = plsc.VectorSubcoreMesh(core_axis_name="c", subcore_axis_name="s")

    @functools.partial(
        pl.kernel,
        out_type=(
            jax.ShapeDtypeStruct((apad * move_w,), jnp.float32),
            jax.ShapeDtypeStruct((apad,), jnp.float32),
            jax.ShapeDtypeStruct((epad,), jnp.float32),
        ),
        mesh=mesh,
        scratch_types=[
            pltpu.VMEM((tn * move_w,), jnp.float32),
            pltpu.VMEM((tn,), jnp.float32),
            pltpu.VMEM((tn,), jnp.float32),
            pltpu.VMEM((apad,), jnp.int32),
            pltpu.VMEM((ept,), jnp.int32),
            pltpu.VMEM((mpt,), jnp.float32),
            pltpu.VMEM((hpt,), jnp.float32),
            pltpu.VMEM((ept,), jnp.float32),
        ],
        compiler_params=pltpu.CompilerParams(needs_layout_passes=False),
    )
    def k(mtab_hbm, htab_hbm, atab_hbm, ally_hbm, adst_hbm,
          mv_hbm, ho_hbm, at_hbm,
          mtab_v, htab_v, atab_v, ally_v, adst_v, mv_v, ho_v, at_v):
        c = lax.axis_index("c")
        s = lax.axis_index("s")
        wid = s * _NC + c
        pltpu.sync_copy(mtab_hbm, mtab_v)
        pltpu.sync_copy(htab_hbm, htab_v)
        pltpu.sync_copy(atab_hbm, atab_v)
        pltpu.sync_copy(ally_hbm, ally_v)
        pltpu.sync_copy(adst_hbm.at[pl.ds(wid * ept, ept)], adst_v)
        iota = lax.iota(jnp.int32, _LANES)

        mbase = wid * mpt
        for kk in range(mpt // _LANES):
            p = jnp.full((_LANES,), mbase + kk * _LANES, jnp.int32) + iota
            j = p // move_w
            cc = p - j * move_w
            a = plsc.load_gather(ally_v, [j])
            v = plsc.load_gather(mtab_v, [a * move_w + cc])
            mv_v[pl.ds(kk * _LANES, _LANES)] = v
        pltpu.sync_copy(mv_v, mv_hbm.at[pl.ds(mbase, mpt)])

        hbase = wid * hpt
        for kk in range(hpt // _LANES):
            p = jnp.full((_LANES,), hbase + kk * _LANES, jnp.int32) + iota
            a = plsc.load_gather(ally_v, [p])
            v = plsc.load_gather(htab_v, [a])
            ho_v[pl.ds(kk * _LANES, _LANES)] = v
        pltpu.sync_copy(ho_v, ho_hbm.at[pl.ds(hbase, hpt)])

        def abody(kk, carry):
            d16 = adst_v[pl.ds(kk * _LANES, _LANES)]
            v = plsc.load_gather(atab_v, [d16])
            at_v[pl.ds(kk * _LANES, _LANES)] = v
            return carry

        lax.fori_loop(0, ept // _LANES, abody, 0)
        pltpu.sync_copy(at_v, at_hbm.at[pl.ds(wid * ept, ept)])

    return k(mtab, htab, atab, ally_pad, adst_pad)


def kernel(node_feature, global_feature, edge_index, attack_edge_index,
           ally_indices, W_msg, b_msg, W_u1, b_u1, W_u2, b_u2, ln_g, ln_b,
           W_glob, b_glob, W_m1, b_m1, W_m2, b_m2, W_h1, b_h1, W_h2, b_h2,
           W_a1, b_a1, W_a2, b_a2):
    n, d = node_feature.shape
    ea = attack_edge_index.shape[1]
    ally = ally_indices.shape[0]
    move_w = W_m2.shape[1]
    nlayers = W_msg.shape[0]
    bn = 1000

    e = edge_index.shape[1]
    rows_per_tile = ((n // _NS + 7) // 8) * 8
    n_pad = rows_per_tile * _NS
    zeros_tile = jnp.zeros((rows_per_tile, d), jnp.float32)

    e_unit = _NW * _CHUNK * 8
    e_pad = ((e + e_unit - 1) // e_unit) * e_unit
    pad = e_pad - e
    pad_ar = jnp.arange(pad, dtype=jnp.int32)
    src1d = jnp.concatenate([edge_index[0], pad_ar % n])
    dst1d = jnp.concatenate([edge_index[1], n + pad_ar % (n_pad - n)])

    x = node_feature
    psum = None
    for l in range(nlayers):
        msg = _tc_msg(x, W_msg[l], b_msg[l], bn)
        aggcat, n_pad = _sc_edge_agg(msg, src1d, dst1d, zeros_tile)
        x, psum = _tc_update(
            x, aggcat[:n], aggcat[n_pad:n_pad + n], W_u1[l], b_u1[l],
            W_u2[l], b_u2[l], ln_g[l], ln_b[l], bn,
            with_psum=(l == nlayers - 1))

    move_n, hold_n, atk_n = _tc_heads(
        x, psum, global_feature, W_glob, b_glob,
        W_m1, b_m1, W_m2, b_m2, W_h1, b_h1, W_h2, b_h2,
        W_a1, b_a1, W_a2, b_a2, bn)

    apad = ((ally + _NW * _LANES - 1) // (_NW * _LANES)) * _NW * _LANES
    epad = ((ea + _NW * _LANES - 1) // (_NW * _LANES)) * _NW * _LANES
    ally_pad = jnp.zeros((apad,), jnp.int32).at[:ally].set(ally_indices)
    adst_pad = jnp.zeros((epad,), jnp.int32).at[:ea].set(attack_edge_index[1])

    mv, ho, at = _sc_heads_gather(
        move_n.reshape(-1), hold_n.reshape(-1), atk_n.reshape(-1),
        ally_pad, adst_pad, move_w)
    return (mv[:ally * move_w].reshape(ally, move_w),
            ho[:ally].reshape(ally, 1),
            at[:ea])


# speedup vs baseline: 1.0665x; 1.0665x over previous
import functools

import jax
import jax.numpy as jnp
from jax import lax
from jax.experimental import pallas as pl
from jax.experimental.pallas import tpu as pltpu
from jax.experimental.pallas import tpu_sc as plsc

_NC = 2
_NS = 16
_NW = _NC * _NS
_LANES = 16
_CHUNK = 120


def _msg_body(x_ref, w_ref, b_ref, o_ref):
    m = jnp.tanh(
        jnp.dot(x_ref[...], w_ref[...], preferred_element_type=jnp.float32)
        + b_ref[...])
    o_ref[...] = jnp.maximum(m, 0.0)


def _tc_msg(x, w, b, bn):
    n, d = x.shape
    return pl.pallas_call(
        _msg_body,
        grid=(n // bn,),
        in_specs=[
            pl.BlockSpec((bn, d), lambda i: (i, 0)),
            pl.BlockSpec((d, d), lambda i: (0, 0)),
            pl.BlockSpec((1, d), lambda i: (0, 0)),
        ],
        out_specs=pl.BlockSpec((bn, d), lambda i: (i, 0)),
        out_shape=jax.ShapeDtypeStruct((n, d), jnp.float32),
    )(x, w, b.reshape(1, d))


def _upd_body(with_psum, with_msg, x_ref, a0_ref, a1_ref, w1_ref, b1_ref,
              w2_ref, b2_ref, g_ref, be_ref, *refs):
    nin = 2 if with_msg else 0
    out_refs = refs[nin:]
    x = x_ref[...]
    agg = a0_ref[...] + a1_ref[...]
    h = jnp.concatenate([x, agg], axis=1)
    h = jnp.tanh(
        jnp.dot(h, w1_ref[...], preferred_element_type=jnp.float32)
        + b1_ref[...])
    h = jnp.dot(h, w2_ref[...], preferred_element_type=jnp.float32) + b2_ref[...]
    mu = jnp.mean(h, axis=-1, keepdims=True)
    c = h - mu
    var = jnp.mean(c * c, axis=-1, keepdims=True)
    xn = g_ref[...] * c / jnp.sqrt(var + 1e-5) + be_ref[...]
    out_refs[0][...] = xn
    if with_msg:
        wm_ref, bm_ref = refs[0], refs[1]
        m = jnp.tanh(
            jnp.dot(xn, wm_ref[...], preferred_element_type=jnp.float32)
            + bm_ref[...])
        out_refs[1][...] = jnp.maximum(m, 0.0)
    if with_psum:
        out_refs[-1][...] = jnp.sum(xn, axis=0)[None, None, :]


def _tc_update(x, a0, a1, w1, b1, w2, b2, ln_g, ln_b, bn, with_psum,
               wmsg=None, bmsg=None):
    n, d = x.shape
    grid = n // bn
    with_msg = wmsg is not None
    out_shape = [jax.ShapeDtypeStruct((n, d), jnp.float32)]
    out_specs = [pl.BlockSpec((bn, d), lambda i: (i, 0))]
    if with_msg:
        out_shape.append(jax.ShapeDtypeStruct((n, d), jnp.float32))
        out_specs.append(pl.BlockSpec((bn, d), lambda i: (i, 0)))
    if with_psum:
        out_shape.append(jax.ShapeDtypeStruct((grid, 1, d), jnp.float32))
        out_specs.append(pl.BlockSpec((1, 1, d), lambda i: (i, 0, 0)))
    in_specs = [
        pl.BlockSpec((bn, d), lambda i: (i, 0)),
        pl.BlockSpec((bn, d), lambda i: (i, 0)),
        pl.BlockSpec((bn, d), lambda i: (i, 0)),
        pl.BlockSpec((2 * d, d), lambda i: (0, 0)),
        pl.BlockSpec((1, d), lambda i: (0, 0)),
        pl.BlockSpec((d, d), lambda i: (0, 0)),
        pl.BlockSpec((1, d), lambda i: (0, 0)),
        pl.BlockSpec((1, d), lambda i: (0, 0)),
        pl.BlockSpec((1, d), lambda i: (0, 0)),
    ]
    args = [x, a0, a1, w1, b1.reshape(1, d), w2, b2.reshape(1, d),
            ln_g.reshape(1, d), ln_b.reshape(1, d)]
    if with_msg:
        in_specs += [pl.BlockSpec((d, d), lambda i: (0, 0)),
                     pl.BlockSpec((1, d), lambda i: (0, 0))]
        args += [wmsg, bmsg.reshape(1, d)]
    res = pl.pallas_call(
        functools.partial(_upd_body, with_psum, with_msg),
        grid=(grid,),
        in_specs=in_specs,
        out_specs=out_specs,
        out_shape=out_shape,
    )(*args)
    xn = res[0]
    msg = res[1] if with_msg else None
    psum = res[-1] if with_psum else None
    return xn, msg, psum


def _heads_body(n_total, x_ref, ps_ref, gf_ref, wg_ref, bg_ref,
                wm1_ref, bm1_ref, wm2_ref, bm2_ref,
                wh1_ref, bh1_ref, wh2_ref, bh2_ref,
                wa1_ref, ba1_ref, wa2_ref, ba2_ref,
                om_ref, oh_ref, oa_ref):
    total = jnp.sum(ps_ref[...], axis=(0, 1))[None, :] * (1.0 / n_total)
    g = jnp.tanh(
        jnp.dot(total, wg_ref[...], preferred_element_type=jnp.float32)
        + bg_ref[...] + gf_ref[...])
    x = x_ref[...]
    feat = jnp.concatenate(
        [x, jnp.broadcast_to(g, (x.shape[0], g.shape[1]))], axis=1)
    hm = jnp.tanh(
        jnp.dot(feat, wm1_ref[...], preferred_element_type=jnp.float32)
        + bm1_ref[...])
    om_ref[...] = jnp.tanh(
        jnp.dot(hm, wm2_ref[...], preferred_element_type=jnp.float32)
        + bm2_ref[...])
    hh = jnp.tanh(
        jnp.dot(feat, wh1_ref[...], preferred_element_type=jnp.float32)
        + bh1_ref[...])
    oh_ref[...] = jnp.tanh(
        jnp.dot(hh, wh2_ref[...], preferred_element_type=jnp.float32)
        + bh2_ref[...])
    ha = jnp.tanh(
        jnp.dot(feat, wa1_ref[...], preferred_element_type=jnp.float32)
        + ba1_ref[...])
    oa_ref[...] = jnp.tanh(
        jnp.dot(ha, wa2_ref[...], preferred_element_type=jnp.float32)
        + ba2_ref[...])


def _tc_heads(x, psum, gf, wg, bg, wm1, bm1, wm2, bm2, wh1, bh1, wh2, bh2,
              wa1, ba1, wa2, ba2, bn):
    n, d = x.shape
    grid = n // bn
    nb = psum.shape[0]
    g_dim = gf.shape[1]
    h_dim = wm1.shape[1]
    dg = d + g_dim
    move_w = wm2.shape[1]
    wspec = lambda a, b: pl.BlockSpec((a, b), lambda i: (0, 0))
    return pl.pallas_call(
        functools.partial(_heads_body, n),
        grid=(grid,),
        in_specs=[
            pl.BlockSpec((bn, d), lambda i: (i, 0)),
            pl.BlockSpec((nb, 1, d), lambda i: (0, 0, 0)),
            wspec(1, g_dim), wspec(d, g_dim), wspec(1, g_dim),
            wspec(dg, h_dim), wspec(1, h_dim), wspec(h_dim, move_w),
            wspec(1, move_w),
            wspec(dg, h_dim), wspec(1, h_dim), wspec(h_dim, 1), wspec(1, 1),
            wspec(dg, h_dim), wspec(1, h_dim), wspec(h_dim, 1), wspec(1, 1),
        ],
        out_specs=[
            pl.BlockSpec((bn, move_w), lambda i: (i, 0)),
            pl.BlockSpec((bn, 1), lambda i: (i, 0)),
            pl.BlockSpec((bn, 1), lambda i: (i, 0)),
        ],
        out_shape=[
            jax.ShapeDtypeStruct((n, move_w), jnp.float32),
            jax.ShapeDtypeStruct((n, 1), jnp.float32),
            jax.ShapeDtypeStruct((n, 1), jnp.float32),
        ],
    )(x, psum, gf, wg, bg.reshape(1, g_dim),
      wm1, bm1.reshape(1, h_dim), wm2, bm2.reshape(1, move_w),
      wh1, bh1.reshape(1, h_dim), wh2, bh2.reshape(1, 1),
      wa1, ba1.reshape(1, h_dim), wa2, ba2.reshape(1, 1))


_NBUF = 3
_IRING = 6


def _sc_edge_agg(msg, src1d, dst1d, zeros_tile):
    n, d = msg.shape
    n_chunks = src1d.shape[0] // _CHUNK
    cpw = n_chunks // _NW
    rows_per_tile = ((n // _NS + 7) // 8) * 8
    n_pad = rows_per_tile * _NS
    mesh = plsc.VectorSubcoreMesh(core_axis_name="c", subcore_axis_name="s")

    @functools.partial(
        pl.kernel,
        out_type=(jax.ShapeDtypeStruct((n_pad, d), jnp.float32),
                  jax.ShapeDtypeStruct((n_pad, d), jnp.float32)),
        mesh=mesh,
        scratch_types=[
            pltpu.VMEM((_IRING, _CHUNK), jnp.int32),
            pltpu.VMEM((_IRING, _CHUNK), jnp.int32),
            pltpu.VMEM((_NBUF, _CHUNK, d), jnp.float32),
            pltpu.VMEM_SHARED((n_pad, d), jnp.float32),
            pltpu.SemaphoreType.DMA((_IRING,)),
            pltpu.SemaphoreType.DMA((_NBUF,)),
            pltpu.SemaphoreType.DMA((_NBUF,)),
        ],
        compiler_params=pltpu.CompilerParams(needs_layout_passes=False),
    )
    def k(msg_hbm, src_hbm, dst_hbm, zero_hbm, out0_hbm, out1_hbm,
          sidx, didx, rows, acc, isem, gsem, ssem):
        c = lax.axis_index("c")
        s = lax.axis_index("s")
        wid = s * _NC + c
        base0 = wid * cpw * _CHUNK

        def prefetch_idx(t):
            @pl.when(t < cpw)
            def _():
                sl = lax.rem(t, _IRING)
                base = base0 + t * _CHUNK
                pltpu.async_copy(src_hbm.at[pl.ds(base, _CHUNK)],
                                 sidx.at[sl], isem.at[sl])
                pltpu.async_copy(dst_hbm.at[pl.ds(base, _CHUNK)],
                                 didx.at[sl], isem.at[sl])

        prefetch_idx(0)
        prefetch_idx(1)
        pltpu.sync_copy(zero_hbm, acc.at[pl.ds(s * rows_per_tile, rows_per_tile)])
        plsc.subcore_barrier()

        def step(j, carry):
            @pl.when(j < cpw)
            def _():
                b = lax.rem(j, _NBUF)
                sl = lax.rem(j, _IRING)

                @pl.when(j >= _NBUF)
                def _():
                    pltpu.make_async_copy(rows.at[b], acc.at[didx.at[0]],
                                          ssem.at[b]).wait()

                pltpu.make_async_copy(src_hbm.at[pl.ds(0, _CHUNK)],
                                      sidx.at[sl], isem.at[sl]).wait()
                pltpu.make_async_copy(dst_hbm.at[pl.ds(0, _CHUNK)],
                                      didx.at[sl], isem.at[sl]).wait()
                pltpu.async_copy(msg_hbm.at[sidx.at[sl]], rows.at[b],
                                 gsem.at[b])
                prefetch_idx(j + 2)

            jj = j - 1

            @pl.when((jj >= 0) & (jj < cpw))
            def _():
                bb = lax.rem(jj, _NBUF)
                sl2 = lax.rem(jj, _IRING)
                pltpu.make_async_copy(msg_hbm.at[sidx.at[0]], rows.at[bb],
                                      gsem.at[bb]).wait()
                pltpu.async_copy(rows.at[bb], acc.at[didx.at[sl2]],
                                 ssem.at[bb], add=True)

            return carry

        lax.fori_loop(0, cpw + 1, step, 0)
        for b in range(_NBUF):
            pltpu.make_async_copy(rows.at[b], acc.at[didx.at[0]],
                                  ssem.at[b]).wait()
        plsc.subcore_barrier()

        @pl.when(c == 0)
        def _():
            pltpu.sync_copy(
                acc.at[pl.ds(s * rows_per_tile, rows_per_tile)],
                out0_hbm.at[pl.ds(s * rows_per_tile, rows_per_tile)])

        @pl.when(c == 1)
        def _():
            pltpu.sync_copy(
                acc.at[pl.ds(s * rows_per_tile, rows_per_tile)],
                out1_hbm.at[pl.ds(s * rows_per_tile, rows_per_tile)])

    return k(msg, src1d, dst1d, zeros_tile), n_pad


def _sc_heads_gather(mtab, htab, atab, ally_pad, adst_pad, move_w):
    tn = htab.shape[0]
    apad = ally_pad.shape[0]
    epad = adst_pad.shape[0]
    ept = epad // _NW
    mpt = apad * move_w // _NW
    hpt = apad // _NW
    mesh = plsc.VectorSubcoreMesh(core_axis_name="c", subcore_axis_name="s")

    @functools.partial(
        pl.kernel,
        out_type=(
            jax.ShapeDtypeStruct((apad * move_w,), jnp.float32),
            jax.ShapeDtypeStruct((apad,), jnp.float32),
            jax.ShapeDtypeStruct((epad,), jnp.float32),
        ),
        mesh=mesh,
        scratch_types=[
            pltpu.VMEM((tn * move_w,), jnp.float32),
            pltpu.VMEM((tn,), jnp.float32),
            pltpu.VMEM((tn,), jnp.float32),
            pltpu.VMEM((apad,), jnp.int32),
            pltpu.VMEM((ept,), jnp.int32),
            pltpu.VMEM((mpt,), jnp.float32),
            pltpu.VMEM((hpt,), jnp.float32),
            pltpu.VMEM((ept,), jnp.float32),
        ],
        compiler_params=pltpu.CompilerParams(needs_layout_passes=False),
    )
    def k(mtab_hbm, htab_hbm, atab_hbm, ally_hbm, adst_hbm,
          mv_hbm, ho_hbm, at_hbm,
          mtab_v, htab_v, atab_v, ally_v, adst_v, mv_v, ho_v, at_v):
        c = lax.axis_index("c")
        s = lax.axis_index("s")
        wid = s * _NC + c
        pltpu.sync_copy(mtab_hbm, mtab_v)
        pltpu.sync_copy(htab_hbm, htab_v)
        pltpu.sync_copy(atab_hbm, atab_v)
        pltpu.sync_copy(ally_hbm, ally_v)
        pltpu.sync_copy(adst_hbm.at[pl.ds(wid * ept, ept)], adst_v)
        iota = lax.iota(jnp.int32, _LANES)

        mbase = wid * mpt
        for kk in range(mpt // _LANES):
            p = jnp.full((_LANES,), mbase + kk * _LANES, jnp.int32) + iota
            j = p // move_w
            cc = p - j * move_w
            a = plsc.load_gather(ally_v, [j])
            v = plsc.load_gather(mtab_v, [a * move_w + cc])
            mv_v[pl.ds(kk * _LANES, _LANES)] = v
        pltpu.sync_copy(mv_v, mv_hbm.at[pl.ds(mbase, mpt)])

        hbase = wid * hpt
        for kk in range(hpt // _LANES):
            p = jnp.full((_LANES,), hbase + kk * _LANES, jnp.int32) + iota
            a = plsc.load_gather(ally_v, [p])
            v = plsc.load_gather(htab_v, [a])
            ho_v[pl.ds(kk * _LANES, _LANES)] = v
        pltpu.sync_copy(ho_v, ho_hbm.at[pl.ds(hbase, hpt)])

        def abody(kk, carry):
            d16 = adst_v[pl.ds(kk * _LANES, _LANES)]
            v = plsc.load_gather(atab_v, [d16])
            at_v[pl.ds(kk * _LANES, _LANES)] = v
            return carry

        lax.fori_loop(0, ept // _LANES, abody, 0)
        pltpu.sync_copy(at_v, at_hbm.at[pl.ds(wid * ept, ept)])

    return k(mtab, htab, atab, ally_pad, adst_pad)


def kernel(node_feature, global_feature, edge_index, attack_edge_index,
           ally_indices, W_msg, b_msg, W_u1, b_u1, W_u2, b_u2, ln_g, ln_b,
           W_glob, b_glob, W_m1, b_m1, W_m2, b_m2, W_h1, b_h1, W_h2, b_h2,
           W_a1, b_a1, W_a2, b_a2):
    n, d = node_feature.shape
    ea = attack_edge_index.shape[1]
    ally = ally_indices.shape[0]
    move_w = W_m2.shape[1]
    nlayers = W_msg.shape[0]
    bn = 1000

    e = edge_index.shape[1]
    rows_per_tile = ((n // _NS + 7) // 8) * 8
    n_pad = rows_per_tile * _NS
    zeros_tile = jnp.zeros((rows_per_tile, d), jnp.float32)

    e_unit = _NW * _CHUNK * 8
    e_pad = ((e + e_unit - 1) // e_unit) * e_unit
    pad = e_pad - e
    pad_ar = jnp.arange(pad, dtype=jnp.int32)
    src1d = jnp.concatenate([edge_index[0], pad_ar % n])
    dst1d = jnp.concatenate([edge_index[1], n + pad_ar % (n_pad - n)])

    x = node_feature
    psum = None
    msg = _tc_msg(x, W_msg[0], b_msg[0], bn)
    for l in range(nlayers):
        (agg0, agg1), n_pad = _sc_edge_agg(msg, src1d, dst1d, zeros_tile)
        last = l == nlayers - 1
        x, msg, psum = _tc_update(
            x, agg0, agg1, W_u1[l], b_u1[l],
            W_u2[l], b_u2[l], ln_g[l], ln_b[l], bn,
            with_psum=last,
            wmsg=None if last else W_msg[l + 1],
            bmsg=None if last else b_msg[l + 1])

    move_n, hold_n, atk_n = _tc_heads(
        x, psum, global_feature, W_glob, b_glob,
        W_m1, b_m1, W_m2, b_m2, W_h1, b_h1, W_h2, b_h2,
        W_a1, b_a1, W_a2, b_a2, bn)

    apad = ((ally + _NW * _LANES - 1) // (_NW * _LANES)) * _NW * _LANES
    epad = ((ea + _NW * _LANES - 1) // (_NW * _LANES)) * _NW * _LANES
    ally_pad = jnp.zeros((apad,), jnp.int32).at[:ally].set(ally_indices)
    adst_pad = jnp.zeros((epad,), jnp.int32).at[:ea].set(attack_edge_index[1])

    mv, ho, at = _sc_heads_gather(
        move_n.reshape(-1), hold_n.reshape(-1), atk_n.reshape(-1),
        ally_pad, adst_pad, move_w)
    return (mv[:ally * move_w].reshape(ally, move_w),
            ho[:ally].reshape(ally, 1),
            at[:ea])
